# baseline (speedup 1.0000x reference)
import functools

import jax
import jax.numpy as jnp
from jax import lax
from jax.experimental import pallas as pl
from jax.experimental.pallas import tpu as pltpu
from jax.experimental.pallas import tpu_sc as plsc

N = 10000
E = 320000
F = 128

NC, NS = 2, 16
NW = NC * NS
BK = 128

NP = 10240
ROWS_PER_TILE = NP // NS

E_PAD = 323584
NBLK = E_PAD // (NW * BK)

_mesh = plsc.VectorSubcoreMesh(core_axis_name="c", subcore_axis_name="s")


@functools.partial(
    pl.kernel,
    mesh=_mesh,
    out_type=jax.ShapeDtypeStruct((NC, NP), jnp.float32),
    scratch_types=[
        pltpu.VMEM((BK,), jnp.int32),
        pltpu.VMEM((BK,), jnp.int32),
        pltpu.VMEM((BK,), jnp.float32),
        pltpu.VMEM_SHARED((NP,), jnp.float32),
        pltpu.SemaphoreType.DMA,
        pltpu.SemaphoreType.DMA,
    ],
)
def _sc_degree(rows_hbm, zeros_hbm, ones_hbm, deg_hbm,
               idx0_v, idx1_v, ones_v, hist_s, sem0, sem1):
    idx = (idx0_v, idx1_v)
    sems = (sem0, sem1)
    cid = lax.axis_index("c")
    sid = lax.axis_index("s")
    wid = cid * NS + sid
    share = sid * ROWS_PER_TILE
    pltpu.sync_copy(ones_hbm, ones_v)
    pltpu.sync_copy(zeros_hbm, hist_s.at[pl.ds(share, ROWS_PER_TILE)])
    plsc.subcore_barrier()

    def ebase(i):
        return pl.multiple_of((wid * NBLK + i) * BK, BK)

    pltpu.async_copy(rows_hbm.at[pl.ds(ebase(0), BK)], idx[0], sems[0])

    def step(i, b):
        o = 1 - b
        pltpu.async_copy(rows_hbm.at[pl.ds(ebase(i + 1), BK)], idx[o],
                         sems[o])
        pltpu.make_async_copy(
            rows_hbm.at[pl.ds(ebase(i), BK)], idx[b], sems[b]).wait()
        pltpu.sync_copy(ones_v, hist_s.at[idx[b]], add=True)

    def body(j, carry):
        step(2 * j, 0)
        step(2 * j + 1, 1)
        return carry

    lax.fori_loop(0, (NBLK - 1) // 2, body, 0)
    pltpu.make_async_copy(
        rows_hbm.at[pl.ds(ebase(NBLK - 1), BK)], idx[0], sems[0]).wait()
    pltpu.sync_copy(ones_v, hist_s.at[idx[0]], add=True)
    plsc.subcore_barrier()
    pltpu.sync_copy(hist_s.at[pl.ds(share, ROWS_PER_TILE)],
                    deg_hbm.at[cid, pl.ds(share, ROWS_PER_TILE)])


@functools.partial(
    pl.kernel,
    mesh=_mesh,
    out_type=jax.ShapeDtypeStruct((NC, NP, F), jnp.float32),
    scratch_types=[
        pltpu.VMEM((BK,), jnp.int32),
        pltpu.VMEM((BK,), jnp.int32),
        pltpu.VMEM((BK, F), jnp.float32),
        pltpu.VMEM_SHARED((NP, F), jnp.float32),
        pltpu.SemaphoreType.DMA,
    ],
)
def _sc_aggregate(h2_hbm, rows_hbm, cols_hbm, zeros_hbm, out_hbm,
                  idxc_v, idxr_v, rows_v, acc_s, sem):
    cid = lax.axis_index("c")
    sid = lax.axis_index("s")
    wid = cid * NS + sid
    share = sid * ROWS_PER_TILE
    pltpu.sync_copy(zeros_hbm, acc_s.at[pl.ds(share, ROWS_PER_TILE)])
    plsc.subcore_barrier()

    def body(i, carry):
        base = pl.multiple_of((wid * NBLK + i) * BK, BK)
        pltpu.sync_copy(cols_hbm.at[pl.ds(base, BK)], idxc_v)
        gather = pltpu.async_copy(h2_hbm.at[idxc_v], rows_v, sem)
        pltpu.sync_copy(rows_hbm.at[pl.ds(base, BK)], idxr_v)
        gather.wait()
        pltpu.sync_copy(rows_v, acc_s.at[idxr_v], add=True)
        return carry

    lax.fori_loop(0, NBLK, body, 0)
    plsc.subcore_barrier()
    pltpu.sync_copy(acc_s.at[pl.ds(share, ROWS_PER_TILE)],
                    out_hbm.at[cid, pl.ds(share, ROWS_PER_TILE)])


def _dis_block(degt_blk):
    return lax.rsqrt(jnp.maximum(degt_blk, 1.0))


def _tc_matmul_body(x_ref, w_ref, h_ref):
    h_ref[...] = jnp.dot(x_ref[...], w_ref[...],
                         preferred_element_type=jnp.float32)


def _tc_scale_body(h_ref, degt_ref, h2_ref):
    h2_ref[...] = h_ref[...] * _dis_block(degt_ref[...])


def _tc_combine_body(q_ref, degt_ref, out_ref):
    dis = _dis_block(degt_ref[...])
    out_ref[...] = (q_ref[0] + q_ref[1]) * dis


_TCB = 1024

_tc_matmul = pl.pallas_call(
    _tc_matmul_body,
    grid=(NP // _TCB,),
    in_specs=[
        pl.BlockSpec((_TCB, F), lambda i: (i, 0)),
        pl.BlockSpec((F, F), lambda i: (0, 0)),
    ],
    out_specs=pl.BlockSpec((_TCB, F), lambda i: (i, 0)),
    out_shape=jax.ShapeDtypeStruct((NP, F), jnp.float32),
)

_tc_scale = pl.pallas_call(
    _tc_scale_body,
    grid=(NP // _TCB,),
    in_specs=[
        pl.BlockSpec((_TCB, F), lambda i: (i, 0)),
        pl.BlockSpec((_TCB, 1), lambda i: (i, 0)),
    ],
    out_specs=pl.BlockSpec((_TCB, F), lambda i: (i, 0)),
    out_shape=jax.ShapeDtypeStruct((NP, F), jnp.float32),
)

_tc_combine = pl.pallas_call(
    _tc_combine_body,
    grid=(NP // _TCB,),
    in_specs=[
        pl.BlockSpec((NC, _TCB, F), lambda i: (0, i, 0)),
        pl.BlockSpec((_TCB, 1), lambda i: (i, 0)),
    ],
    out_specs=pl.BlockSpec((_TCB, F), lambda i: (i, 0)),
    out_shape=jax.ShapeDtypeStruct((NP, F), jnp.float32),
)


@jax.jit
def kernel(x, edge_index, weight):
    row = edge_index[0]
    col = edge_index[1]
    rp = jnp.pad(row, (0, E_PAD - E), constant_values=N)
    cp = jnp.pad(col, (0, E_PAD - E), constant_values=N)
    xp = jnp.pad(x, ((0, NP - N), (0, 0)))

    zeros_deg = jnp.zeros((ROWS_PER_TILE,), jnp.float32)
    ones_blk = jnp.ones((BK,), jnp.float32)
    degp = _sc_degree(rp, zeros_deg, ones_blk)
    h = _tc_matmul(xp, weight)
    degt = (degp[0] + degp[1])[:, None]

    h2 = _tc_scale(h, degt)

    zeros_rows = jnp.zeros((ROWS_PER_TILE, F), jnp.float32)
    q = _sc_aggregate(h2, rp, cp, zeros_rows)

    out = _tc_combine(q, degt)
    return out[:N]


# speedup vs baseline: 1.0824x; 1.0824x over previous
import functools

import jax
import jax.numpy as jnp
from jax import lax
from jax.experimental import pallas as pl
from jax.experimental.pallas import tpu as pltpu
from jax.experimental.pallas import tpu_sc as plsc

N = 10000
E = 320000
F = 128

NC, NS = 2, 16
NW = NC * NS
BK = 128

NP = 10240
ROWS_PER_TILE = NP // NS

E_PAD = 323584
NBLK = E_PAD // (NW * BK)

_mesh = plsc.VectorSubcoreMesh(core_axis_name="c", subcore_axis_name="s")


@functools.partial(
    pl.kernel,
    mesh=_mesh,
    out_type=jax.ShapeDtypeStruct((NC, NP), jnp.float32),
    scratch_types=[
        pltpu.VMEM((BK,), jnp.int32),
        pltpu.VMEM((BK,), jnp.int32),
        pltpu.VMEM((BK,), jnp.float32),
        pltpu.VMEM_SHARED((NP,), jnp.float32),
        pltpu.SemaphoreType.DMA,
        pltpu.SemaphoreType.DMA,
    ],
)
def _sc_degree(rows_hbm, zeros_hbm, ones_hbm, deg_hbm,
               idx0_v, idx1_v, ones_v, hist_s, sem0, sem1):
    idx = (idx0_v, idx1_v)
    sems = (sem0, sem1)
    cid = lax.axis_index("c")
    sid = lax.axis_index("s")
    wid = cid * NS + sid
    share = sid * ROWS_PER_TILE
    pltpu.sync_copy(ones_hbm, ones_v)
    pltpu.sync_copy(zeros_hbm, hist_s.at[pl.ds(share, ROWS_PER_TILE)])
    plsc.subcore_barrier()

    def ebase(i):
        return pl.multiple_of((wid * NBLK + i) * BK, BK)

    pltpu.async_copy(rows_hbm.at[pl.ds(ebase(0), BK)], idx[0], sems[0])

    def step(i, b):
        o = 1 - b
        pltpu.async_copy(rows_hbm.at[pl.ds(ebase(i + 1), BK)], idx[o],
                         sems[o])
        pltpu.make_async_copy(
            rows_hbm.at[pl.ds(ebase(i), BK)], idx[b], sems[b]).wait()
        pltpu.sync_copy(ones_v, hist_s.at[idx[b]], add=True)

    def body(j, carry):
        step(2 * j, 0)
        step(2 * j + 1, 1)
        return carry

    lax.fori_loop(0, (NBLK - 1) // 2, body, 0)
    pltpu.make_async_copy(
        rows_hbm.at[pl.ds(ebase(NBLK - 1), BK)], idx[0], sems[0]).wait()
    pltpu.sync_copy(ones_v, hist_s.at[idx[0]], add=True)
    plsc.subcore_barrier()
    pltpu.sync_copy(hist_s.at[pl.ds(share, ROWS_PER_TILE)],
                    deg_hbm.at[cid, pl.ds(share, ROWS_PER_TILE)])


@functools.partial(
    pl.kernel,
    mesh=_mesh,
    out_type=jax.ShapeDtypeStruct((NC, NP, F), jnp.float32),
    scratch_types=[
        pltpu.VMEM((BK,), jnp.int32),
        pltpu.VMEM((BK,), jnp.int32),
        pltpu.VMEM((BK,), jnp.int32),
        pltpu.VMEM((BK, F), jnp.float32),
        pltpu.VMEM_SHARED((NP, F), jnp.float32),
        pltpu.SemaphoreType.DMA,
        pltpu.SemaphoreType.DMA,
        pltpu.SemaphoreType.DMA,
    ],
)
def _sc_aggregate(h2_hbm, rows_hbm, cols_hbm, zeros_hbm, out_hbm,
                  idxc0_v, idxc1_v, idxr_v, rows_v, acc_s,
                  semc0, semc1, semg):
    idxc = (idxc0_v, idxc1_v)
    semc = (semc0, semc1)
    cid = lax.axis_index("c")
    sid = lax.axis_index("s")
    wid = cid * NS + sid
    share = sid * ROWS_PER_TILE
    pltpu.sync_copy(zeros_hbm, acc_s.at[pl.ds(share, ROWS_PER_TILE)])
    plsc.subcore_barrier()

    def ebase(i):
        return pl.multiple_of((wid * NBLK + i) * BK, BK)

    pltpu.async_copy(cols_hbm.at[pl.ds(ebase(0), BK)], idxc[0], semc[0])

    def step(i, b):
        o = 1 - b
        pltpu.async_copy(cols_hbm.at[pl.ds(ebase(i + 1), BK)], idxc[o],
                         semc[o])
        pltpu.make_async_copy(
            cols_hbm.at[pl.ds(ebase(i), BK)], idxc[b], semc[b]).wait()
        gather = pltpu.async_copy(h2_hbm.at[idxc[b]], rows_v, semg)
        pltpu.sync_copy(rows_hbm.at[pl.ds(ebase(i), BK)], idxr_v)
        gather.wait()
        pltpu.sync_copy(rows_v, acc_s.at[idxr_v], add=True)

    def body(j, carry):
        step(2 * j, 0)
        step(2 * j + 1, 1)
        return carry

    lax.fori_loop(0, (NBLK - 1) // 2, body, 0)
    pltpu.make_async_copy(
        cols_hbm.at[pl.ds(ebase(NBLK - 1), BK)], idxc[0], semc[0]).wait()
    gather = pltpu.async_copy(h2_hbm.at[idxc[0]], rows_v, semg)
    pltpu.sync_copy(rows_hbm.at[pl.ds(ebase(NBLK - 1), BK)], idxr_v)
    gather.wait()
    pltpu.sync_copy(rows_v, acc_s.at[idxr_v], add=True)
    plsc.subcore_barrier()
    pltpu.sync_copy(acc_s.at[pl.ds(share, ROWS_PER_TILE)],
                    out_hbm.at[cid, pl.ds(share, ROWS_PER_TILE)])


def _dis_block(degt_blk):
    return lax.rsqrt(jnp.maximum(degt_blk, 1.0))


def _tc_scale_body(x_ref, w_ref, degt_ref, h2_ref):
    dis = _dis_block(degt_ref[...])
    h2_ref[...] = jnp.dot(x_ref[...], w_ref[...],
                          preferred_element_type=jnp.float32) * dis


def _tc_combine_body(q_ref, degt_ref, out_ref):
    dis = _dis_block(degt_ref[...])
    out_ref[...] = (q_ref[0] + q_ref[1]) * dis


_TCB = 1024

_tc_scale = pl.pallas_call(
    _tc_scale_body,
    grid=(NP // _TCB,),
    in_specs=[
        pl.BlockSpec((_TCB, F), lambda i: (i, 0)),
        pl.BlockSpec((F, F), lambda i: (0, 0)),
        pl.BlockSpec((_TCB, 1), lambda i: (i, 0)),
    ],
    out_specs=pl.BlockSpec((_TCB, F), lambda i: (i, 0)),
    out_shape=jax.ShapeDtypeStruct((NP, F), jnp.float32),
)

_tc_combine = pl.pallas_call(
    _tc_combine_body,
    grid=(NP // _TCB,),
    in_specs=[
        pl.BlockSpec((NC, _TCB, F), lambda i: (0, i, 0)),
        pl.BlockSpec((_TCB, 1), lambda i: (i, 0)),
    ],
    out_specs=pl.BlockSpec((_TCB, F), lambda i: (i, 0)),
    out_shape=jax.ShapeDtypeStruct((NP, F), jnp.float32),
)


@jax.jit
def kernel(x, edge_index, weight):
    row = edge_index[0]
    col = edge_index[1]
    rp = jnp.pad(row, (0, E_PAD - E), constant_values=N)
    cp = jnp.pad(col, (0, E_PAD - E), constant_values=N)
    xp = jnp.pad(x, ((0, NP - N), (0, 0)))

    zeros_deg = jnp.zeros((ROWS_PER_TILE,), jnp.float32)
    ones_blk = jnp.ones((BK,), jnp.float32)
    degp = _sc_degree(rp, zeros_deg, ones_blk)
    degt = (degp[0] + degp[1])[:, None]

    h2 = _tc_scale(xp, weight, degt)

    zeros_rows = jnp.zeros((ROWS_PER_TILE, F), jnp.float32)
    q = _sc_aggregate(h2, rp, cp, zeros_rows)

    out = _tc_combine(q, degt)
    return out[:N]


# speedup vs baseline: 1.1838x; 1.0937x over previous
import functools

import jax
import jax.numpy as jnp
from jax import lax
from jax.experimental import pallas as pl
from jax.experimental.pallas import tpu as pltpu
from jax.experimental.pallas import tpu_sc as plsc

N = 10000
E = 320000
F = 128

NC, NS = 2, 16
NW = NC * NS
BK = 128

NP = 10240
ROWS_PER_TILE = NP // NS

E_PAD = 323584
NBLK = E_PAD // (NW * BK)

_mesh = plsc.VectorSubcoreMesh(core_axis_name="c", subcore_axis_name="s")


@functools.partial(
    pl.kernel,
    mesh=_mesh,
    out_type=jax.ShapeDtypeStruct((NC, NP), jnp.float32),
    scratch_types=[
        pltpu.VMEM((BK,), jnp.int32),
        pltpu.VMEM((BK,), jnp.int32),
        pltpu.VMEM((BK,), jnp.float32),
        pltpu.VMEM_SHARED((NP,), jnp.float32),
        pltpu.SemaphoreType.DMA,
        pltpu.SemaphoreType.DMA,
    ],
)
def _sc_degree(rows_hbm, zeros_hbm, ones_hbm, deg_hbm,
               idx0_v, idx1_v, ones_v, hist_s, sem0, sem1):
    idx = (idx0_v, idx1_v)
    sems = (sem0, sem1)
    cid = lax.axis_index("c")
    sid = lax.axis_index("s")
    wid = cid * NS + sid
    share = sid * ROWS_PER_TILE
    pltpu.sync_copy(ones_hbm, ones_v)
    pltpu.sync_copy(zeros_hbm, hist_s.at[pl.ds(share, ROWS_PER_TILE)])
    plsc.subcore_barrier()

    def ebase(i):
        return pl.multiple_of((wid * NBLK + i) * BK, BK)

    pltpu.async_copy(rows_hbm.at[pl.ds(ebase(0), BK)], idx[0], sems[0])

    def step(i, b):
        o = 1 - b
        pltpu.async_copy(rows_hbm.at[pl.ds(ebase(i + 1), BK)], idx[o],
                         sems[o])
        pltpu.make_async_copy(
            rows_hbm.at[pl.ds(ebase(i), BK)], idx[b], sems[b]).wait()
        pltpu.sync_copy(ones_v, hist_s.at[idx[b]], add=True)

    def body(j, carry):
        step(2 * j, 0)
        step(2 * j + 1, 1)
        return carry

    lax.fori_loop(0, (NBLK - 1) // 2, body, 0)
    pltpu.make_async_copy(
        rows_hbm.at[pl.ds(ebase(NBLK - 1), BK)], idx[0], sems[0]).wait()
    pltpu.sync_copy(ones_v, hist_s.at[idx[0]], add=True)
    plsc.subcore_barrier()
    pltpu.sync_copy(hist_s.at[pl.ds(share, ROWS_PER_TILE)],
                    deg_hbm.at[cid, pl.ds(share, ROWS_PER_TILE)])


@functools.partial(
    pl.kernel,
    mesh=_mesh,
    out_type=jax.ShapeDtypeStruct((NC, NP, F), jnp.float32),
    scratch_types=[
        pltpu.VMEM((BK,), jnp.int32),
        pltpu.VMEM((BK,), jnp.int32),
        pltpu.VMEM((BK,), jnp.int32),
        pltpu.VMEM((BK,), jnp.int32),
        pltpu.VMEM((BK, F), jnp.float32),
        pltpu.VMEM((BK, F), jnp.float32),
        pltpu.VMEM_SHARED((NP, F), jnp.float32),
        pltpu.SemaphoreType.DMA,
        pltpu.SemaphoreType.DMA,
        pltpu.SemaphoreType.DMA,
        pltpu.SemaphoreType.DMA,
    ],
)
def _sc_aggregate(h2_hbm, rows_hbm, cols_hbm, zeros_hbm, out_hbm,
                  idxc0_v, idxc1_v, idxr0_v, idxr1_v, rows0_v, rows1_v,
                  acc_s, semc0, semc1, semg0, semg1):
    idxc = (idxc0_v, idxc1_v)
    idxr = (idxr0_v, idxr1_v)
    rows = (rows0_v, rows1_v)
    semc = (semc0, semc1)
    semg = (semg0, semg1)
    cid = lax.axis_index("c")
    sid = lax.axis_index("s")
    wid = cid * NS + sid
    share = sid * ROWS_PER_TILE
    pltpu.sync_copy(zeros_hbm, acc_s.at[pl.ds(share, ROWS_PER_TILE)])
    plsc.subcore_barrier()

    def ebase(i):
        return pl.multiple_of((wid * NBLK + i) * BK, BK)

    pltpu.async_copy(cols_hbm.at[pl.ds(ebase(0), BK)], idxc[0], semc[0])
    pltpu.async_copy(cols_hbm.at[pl.ds(ebase(1), BK)], idxc[1], semc[1])

    def fire(i, b):
        pltpu.make_async_copy(
            cols_hbm.at[pl.ds(ebase(i), BK)], idxc[b], semc[b]).wait()
        g = pltpu.async_copy(h2_hbm.at[idxc[b]], rows[b], semg[b])
        pltpu.sync_copy(rows_hbm.at[pl.ds(ebase(i), BK)], idxr[b])
        return g

    def drain(i, b):
        def refill():
            pltpu.async_copy(cols_hbm.at[pl.ds(ebase(i + 2), BK)], idxc[b],
                             semc[b])
        pl.when(i + 2 < NBLK)(refill)
        pltpu.sync_copy(rows[b], acc_s.at[idxr[b]], add=True)

    def body(j, carry):
        g0 = fire(2 * j, 0)
        g1 = fire(2 * j + 1, 1)
        g0.wait()
        drain(2 * j, 0)
        g1.wait()
        drain(2 * j + 1, 1)
        return carry

    lax.fori_loop(0, (NBLK - 1) // 2, body, 0)
    pltpu.make_async_copy(
        cols_hbm.at[pl.ds(ebase(NBLK - 1), BK)], idxc[0], semc[0]).wait()
    gather = pltpu.async_copy(h2_hbm.at[idxc[0]], rows[0], semg[0])
    pltpu.sync_copy(rows_hbm.at[pl.ds(ebase(NBLK - 1), BK)], idxr[0])
    gather.wait()
    pltpu.sync_copy(rows[0], acc_s.at[idxr[0]], add=True)
    plsc.subcore_barrier()
    pltpu.sync_copy(acc_s.at[pl.ds(share, ROWS_PER_TILE)],
                    out_hbm.at[cid, pl.ds(share, ROWS_PER_TILE)])


def _dis_block(degt_blk):
    return lax.rsqrt(jnp.maximum(degt_blk, 1.0))


def _tc_scale_body(x_ref, w_ref, degt_ref, h2_ref):
    dis = _dis_block(degt_ref[...])
    h2_ref[...] = jnp.dot(x_ref[...], w_ref[...],
                          preferred_element_type=jnp.float32) * dis


def _tc_combine_body(q_ref, degt_ref, out_ref):
    dis = _dis_block(degt_ref[...])
    out_ref[...] = (q_ref[0] + q_ref[1]) * dis


_TCB = 1024

_tc_scale = pl.pallas_call(
    _tc_scale_body,
    grid=(NP // _TCB,),
    in_specs=[
        pl.BlockSpec((_TCB, F), lambda i: (i, 0)),
        pl.BlockSpec((F, F), lambda i: (0, 0)),
        pl.BlockSpec((_TCB, 1), lambda i: (i, 0)),
    ],
    out_specs=pl.BlockSpec((_TCB, F), lambda i: (i, 0)),
    out_shape=jax.ShapeDtypeStruct((NP, F), jnp.float32),
)

_tc_combine = pl.pallas_call(
    _tc_combine_body,
    grid=(NP // _TCB,),
    in_specs=[
        pl.BlockSpec((NC, _TCB, F), lambda i: (0, i, 0)),
        pl.BlockSpec((_TCB, 1), lambda i: (i, 0)),
    ],
    out_specs=pl.BlockSpec((_TCB, F), lambda i: (i, 0)),
    out_shape=jax.ShapeDtypeStruct((NP, F), jnp.float32),
)


@jax.jit
def kernel(x, edge_index, weight):
    row = edge_index[0]
    col = edge_index[1]
    rp = jnp.pad(row, (0, E_PAD - E), constant_values=N)
    cp = jnp.pad(col, (0, E_PAD - E), constant_values=N)
    xp = jnp.pad(x, ((0, NP - N), (0, 0)))

    zeros_deg = jnp.zeros((ROWS_PER_TILE,), jnp.float32)
    ones_blk = jnp.ones((BK,), jnp.float32)
    degp = _sc_degree(rp, zeros_deg, ones_blk)
    degt = (degp[0] + degp[1])[:, None]

    h2 = _tc_scale(xp, weight, degt)

    zeros_rows = jnp.zeros((ROWS_PER_TILE, F), jnp.float32)
    q = _sc_aggregate(h2, rp, cp, zeros_rows)

    out = _tc_combine(q, degt)
    return out[:N]


# speedup vs baseline: 1.2948x; 1.0938x over previous
import functools

import jax
import jax.numpy as jnp
from jax import lax
from jax.experimental import pallas as pl
from jax.experimental.pallas import tpu as pltpu
from jax.experimental.pallas import tpu_sc as plsc

N = 10000
E = 320000
F = 128

NC, NS = 2, 16
NW = NC * NS
BK = 128

NP = 10240
ROWS_PER_TILE = NP // NS

E_PAD = 323584
NBLK = E_PAD // (NW * BK)

_mesh = plsc.VectorSubcoreMesh(core_axis_name="c", subcore_axis_name="s")


@functools.partial(
    pl.kernel,
    mesh=_mesh,
    out_type=jax.ShapeDtypeStruct((NC, NP), jnp.float32),
    scratch_types=[
        pltpu.VMEM((BK,), jnp.int32),
        pltpu.VMEM((BK,), jnp.int32),
        pltpu.VMEM((BK,), jnp.float32),
        pltpu.VMEM_SHARED((NP,), jnp.float32),
        pltpu.SemaphoreType.DMA,
        pltpu.SemaphoreType.DMA,
    ],
)
def _sc_degree(rows_hbm, zeros_hbm, ones_hbm, deg_hbm,
               idx0_v, idx1_v, ones_v, hist_s, sem0, sem1):
    idx = (idx0_v, idx1_v)
    sems = (sem0, sem1)
    cid = lax.axis_index("c")
    sid = lax.axis_index("s")
    wid = cid * NS + sid
    share = sid * ROWS_PER_TILE
    pltpu.sync_copy(ones_hbm, ones_v)
    pltpu.sync_copy(zeros_hbm, hist_s.at[pl.ds(share, ROWS_PER_TILE)])
    plsc.subcore_barrier()

    def ebase(i):
        return pl.multiple_of((wid * NBLK + i) * BK, BK)

    pltpu.async_copy(rows_hbm.at[pl.ds(ebase(0), BK)], idx[0], sems[0])

    def step(i, b):
        o = 1 - b
        pltpu.async_copy(rows_hbm.at[pl.ds(ebase(i + 1), BK)], idx[o],
                         sems[o])
        pltpu.make_async_copy(
            rows_hbm.at[pl.ds(ebase(i), BK)], idx[b], sems[b]).wait()
        pltpu.sync_copy(ones_v, hist_s.at[idx[b]], add=True)

    def body(j, carry):
        step(2 * j, 0)
        step(2 * j + 1, 1)
        return carry

    lax.fori_loop(0, (NBLK - 1) // 2, body, 0)
    pltpu.make_async_copy(
        rows_hbm.at[pl.ds(ebase(NBLK - 1), BK)], idx[0], sems[0]).wait()
    pltpu.sync_copy(ones_v, hist_s.at[idx[0]], add=True)
    plsc.subcore_barrier()
    pltpu.sync_copy(hist_s.at[pl.ds(share, ROWS_PER_TILE)],
                    deg_hbm.at[cid, pl.ds(share, ROWS_PER_TILE)])


@functools.partial(
    pl.kernel,
    mesh=_mesh,
    out_type=jax.ShapeDtypeStruct((NC, NP, F), jnp.float32),
    scratch_types=[
        pltpu.VMEM((BK,), jnp.int32),
        pltpu.VMEM((BK,), jnp.int32),
        pltpu.VMEM((BK,), jnp.int32),
        pltpu.VMEM((BK,), jnp.int32),
        pltpu.VMEM((BK, F), jnp.float32),
        pltpu.VMEM((BK, F), jnp.float32),
        pltpu.VMEM_SHARED((NP, F), jnp.float32),
        pltpu.SemaphoreType.DMA,
        pltpu.SemaphoreType.DMA,
        pltpu.SemaphoreType.DMA,
        pltpu.SemaphoreType.DMA,
    ],
)
def _sc_aggregate(h2_hbm, rows_hbm, cols_hbm, zeros_hbm, out_hbm,
                  idxc0_v, idxc1_v, idxr0_v, idxr1_v, rows0_v, rows1_v,
                  acc_s, semc0, semc1, semg0, semg1):
    idxc = (idxc0_v, idxc1_v)
    idxr = (idxr0_v, idxr1_v)
    rows = (rows0_v, rows1_v)
    semc = (semc0, semc1)
    semg = (semg0, semg1)
    cid = lax.axis_index("c")
    sid = lax.axis_index("s")
    wid = cid * NS + sid
    share = sid * ROWS_PER_TILE
    pltpu.sync_copy(zeros_hbm, acc_s.at[pl.ds(share, ROWS_PER_TILE)])
    plsc.subcore_barrier()

    def ebase(i):
        return pl.multiple_of((wid * NBLK + i) * BK, BK)

    pltpu.async_copy(cols_hbm.at[pl.ds(ebase(0), BK)], idxc[0], semc[0])
    pltpu.async_copy(cols_hbm.at[pl.ds(ebase(1), BK)], idxc[1], semc[1])

    def gwait(b):
        pltpu.make_async_copy(h2_hbm.at[idxc[b]], rows[b], semg[b]).wait()

    def fire(i, b):
        pltpu.make_async_copy(
            cols_hbm.at[pl.ds(ebase(i), BK)], idxc[b], semc[b]).wait()
        g = pltpu.async_copy(h2_hbm.at[idxc[b]], rows[b], semg[b])
        pltpu.sync_copy(rows_hbm.at[pl.ds(ebase(i), BK)], idxr[b])
        return g

    def drain(i, b):
        def refill():
            pltpu.async_copy(cols_hbm.at[pl.ds(ebase(i + 2), BK)], idxc[b],
                             semc[b])
        pl.when(i + 2 < NBLK)(refill)
        pltpu.sync_copy(rows[b], acc_s.at[idxr[b]], add=True)

    fire(0, 0)

    def body(j, carry):
        fire(2 * j + 1, 1)
        gwait(0)
        drain(2 * j, 0)
        fire(2 * j + 2, 0)
        gwait(1)
        drain(2 * j + 1, 1)
        return carry

    lax.fori_loop(0, (NBLK - 1) // 2, body, 0)
    gwait(0)
    pltpu.sync_copy(rows[0], acc_s.at[idxr[0]], add=True)
    plsc.subcore_barrier()
    pltpu.sync_copy(acc_s.at[pl.ds(share, ROWS_PER_TILE)],
                    out_hbm.at[cid, pl.ds(share, ROWS_PER_TILE)])


def _dis_block(degt_blk):
    return lax.rsqrt(jnp.maximum(degt_blk, 1.0))


def _tc_scale_body(x_ref, w_ref, degt_ref, h2_ref):
    dis = _dis_block(degt_ref[...])
    h2_ref[...] = jnp.dot(x_ref[...], w_ref[...],
                          preferred_element_type=jnp.float32) * dis


def _tc_combine_body(q_ref, degt_ref, out_ref):
    dis = _dis_block(degt_ref[...])
    out_ref[...] = (q_ref[0] + q_ref[1]) * dis


_TCB = 1024

_tc_scale = pl.pallas_call(
    _tc_scale_body,
    grid=(NP // _TCB,),
    in_specs=[
        pl.BlockSpec((_TCB, F), lambda i: (i, 0)),
        pl.BlockSpec((F, F), lambda i: (0, 0)),
        pl.BlockSpec((_TCB, 1), lambda i: (i, 0)),
    ],
    out_specs=pl.BlockSpec((_TCB, F), lambda i: (i, 0)),
    out_shape=jax.ShapeDtypeStruct((NP, F), jnp.float32),
)

_tc_combine = pl.pallas_call(
    _tc_combine_body,
    grid=(NP // _TCB,),
    in_specs=[
        pl.BlockSpec((NC, _TCB, F), lambda i: (0, i, 0)),
        pl.BlockSpec((_TCB, 1), lambda i: (i, 0)),
    ],
    out_specs=pl.BlockSpec((_TCB, F), lambda i: (i, 0)),
    out_shape=jax.ShapeDtypeStruct((NP, F), jnp.float32),
)


@jax.jit
def kernel(x, edge_index, weight):
    row = edge_index[0]
    col = edge_index[1]
    rp = jnp.pad(row, (0, E_PAD - E), constant_values=N)
    cp = jnp.pad(col, (0, E_PAD - E), constant_values=N)
    xp = jnp.pad(x, ((0, NP - N), (0, 0)))

    zeros_deg = jnp.zeros((ROWS_PER_TILE,), jnp.float32)
    ones_blk = jnp.ones((BK,), jnp.float32)
    degp = _sc_degree(rp, zeros_deg, ones_blk)
    degt = (degp[0] + degp[1])[:, None]

    h2 = _tc_scale(xp, weight, degt)

    zeros_rows = jnp.zeros((ROWS_PER_TILE, F), jnp.float32)
    q = _sc_aggregate(h2, rp, cp, zeros_rows)

    out = _tc_combine(q, degt)
    return out[:N]
